# baseline (speedup 1.0000x reference)
import functools

import jax
import jax.numpy as jnp
from jax import lax
from jax.experimental import pallas as pl
from jax.experimental.pallas import tpu as pltpu
from jax.experimental.pallas import tpu_sc as plsc

H = 4
HD = 32
HID = 128
FH = 4
WIN = 10

_NC = 2
_NS = 16
_NW = _NC * _NS

def _mesh():
    return plsc.VectorSubcoreMesh(
        core_axis_name="c", subcore_axis_name="s",
        num_cores=_NC, num_subcores=_NS)


def _sc_gather(table, idx, chunk):
    M = idx.shape[0]
    R, D = table.shape
    dt = table.dtype
    per_w = M // _NW
    n_pairs = per_w // (2 * chunk)

    @functools.partial(
        pl.kernel,
        mesh=_mesh(),
        out_type=jax.ShapeDtypeStruct((M, D), dt),
        scratch_types=[
            pltpu.VMEM((chunk,), jnp.int32),
            pltpu.VMEM((chunk,), jnp.int32),
            pltpu.VMEM((chunk, D), dt),
            pltpu.VMEM((chunk, D), dt),
            pltpu.SemaphoreType.DMA,
            pltpu.SemaphoreType.DMA,
        ],
    )
    def k(table_hbm, idx_hbm, out_hbm, idx_v0, idx_v1, rows_v0, rows_v1,
          sem0, sem1):
        wid = lax.axis_index("s") * _NC + lax.axis_index("c")
        base = wid * per_w

        def body(j, carry):
            off0 = base + (2 * j) * chunk
            off1 = off0 + chunk
            pltpu.sync_copy(idx_hbm.at[pl.ds(off0, chunk)], idx_v0)
            cp0 = pltpu.async_copy(table_hbm.at[idx_v0], rows_v0, sem0)
            pltpu.sync_copy(idx_hbm.at[pl.ds(off1, chunk)], idx_v1)
            cp1 = pltpu.async_copy(table_hbm.at[idx_v1], rows_v1, sem1)
            cp0.wait()
            pltpu.sync_copy(rows_v0, out_hbm.at[pl.ds(off0, chunk)])
            cp1.wait()
            pltpu.sync_copy(rows_v1, out_hbm.at[pl.ds(off1, chunk)])
            return carry

        lax.fori_loop(0, n_pairs, body, 0)

    return k(table, idx)


def _sc_segsum(upd, idx, nrows, chunk):
    M, D = upd.shape
    half = M // _NC
    per_w = half // _NS
    n_chunks = per_w // chunk
    zeros = jnp.zeros((nrows, D), jnp.float32)

    @functools.partial(
        pl.kernel,
        mesh=_mesh(),
        out_type=jax.ShapeDtypeStruct((_NC, nrows, D), jnp.float32),
        scratch_types=[
            pltpu.VMEM((chunk,), jnp.int32),
            pltpu.VMEM((chunk, D), jnp.float32),
            pltpu.VMEM_SHARED((nrows, D), jnp.float32),
        ],
    )
    def k(upd_hbm, idx_hbm, zeros_hbm, out_hbm, idx_v, upd_v, acc):
        c = lax.axis_index("c")
        s = lax.axis_index("s")
        base = c * half + s * per_w

        @pl.when(s == 0)
        def _():
            pltpu.sync_copy(zeros_hbm, acc)

        plsc.subcore_barrier()

        def body(j, carry):
            off = base + j * chunk
            pltpu.sync_copy(idx_hbm.at[pl.ds(off, chunk)], idx_v)
            pltpu.sync_copy(upd_hbm.at[pl.ds(off, chunk)], upd_v)
            pltpu.sync_copy(upd_v, acc.at[idx_v], add=True)
            return carry

        lax.fori_loop(0, n_chunks, body, 0)
        plsc.subcore_barrier()

        @pl.when(s == 0)
        def _():
            pltpu.sync_copy(acc, out_hbm.at[c])

    return k(upd, idx, zeros)


def _sc_scatter_msgs(msgs, idx, nrows, chunk):
    M = msgs.shape[0]
    half = M // _NC
    per_w = half // _NS
    n_chunks = per_w // chunk
    zeros = jnp.zeros((nrows, HID), jnp.float32)

    @functools.partial(
        pl.kernel,
        mesh=_mesh(),
        out_type=jax.ShapeDtypeStruct((_NC, nrows, WIN, HID), jnp.float32),
        scratch_types=[
            pltpu.VMEM((chunk,), jnp.int32),
            pltpu.VMEM((chunk, HID), jnp.float32),
            pltpu.VMEM_SHARED((nrows, HID), jnp.float32),
        ],
    )
    def k(msgs_hbm, idx_hbm, zeros_hbm, out_hbm, idx_v, upd_v, acc):
        c = lax.axis_index("c")
        s = lax.axis_index("s")
        base = c * half + s * per_w

        for t in range(WIN):
            @pl.when(s == 0)
            def _():
                pltpu.sync_copy(zeros_hbm, acc)

            plsc.subcore_barrier()

            def body(j, carry):
                off = base + j * chunk
                pltpu.sync_copy(idx_hbm.at[pl.ds(off, chunk)], idx_v)
                pltpu.sync_copy(
                    msgs_hbm.at[pl.ds(off, chunk), pl.ds(t * HID, HID)], upd_v)
                pltpu.sync_copy(upd_v, acc.at[idx_v], add=True)
                return carry

            lax.fori_loop(0, n_chunks, body, 0)
            plsc.subcore_barrier()

            @pl.when(s == 0)
            def _():
                pltpu.sync_copy(acc, out_hbm.at[c, :, t])

    return k(msgs, idx, zeros)


def _ln(x, g, b):
    m = x.mean(-1, keepdims=True)
    v = ((x - m) ** 2).mean(-1, keepdims=True)
    return g * (x - m) / jnp.sqrt(v + 1e-5) + b


def _gatv2_all(hs, src, dst, edge_emb, p, pre, num):
    TH = WIN * H
    xl = jnp.einsum('nwd,kd->nwk', hs, p[pre + '_ll_w']) + p[pre + '_ll_b']
    xr = jnp.einsum('nwd,kd->nwk', hs, p[pre + '_lr_w']) + p[pre + '_lr_b']
    XL = xl.reshape(num, WIN * HID)
    XR = xr.reshape(num, WIN * HID)
    XJ = _sc_gather(XL, src, 40)
    XI = _sc_gather(XR, dst, 40)
    e = edge_emb @ p[pre + '_le_w'].T
    m = jax.nn.leaky_relu(
        XJ.reshape(-1, WIN, H, HD) + XI.reshape(-1, WIN, H, HD)
        + e.reshape(-1, 1, H, HD), 0.2)
    L = jnp.sum(m * p[pre + '_att'][:, None], axis=-1).reshape(-1, TH)
    ne = L.shape[0]
    P = jnp.exp(L - jnp.max(L))
    Pp = jnp.zeros((ne, 128), jnp.float32).at[:, :TH].set(P)
    S = _sc_segsum(Pp, dst, num, 80).sum(0)
    msgs = (XJ.reshape(-1, WIN, H, HD)
            * P.reshape(-1, WIN, H, 1)).reshape(-1, WIN * HID)
    out = _sc_scatter_msgs(msgs, dst, num, 80).sum(0)
    denom = S[:, :TH].reshape(num, WIN, H)
    out = out.reshape(num, WIN, H, HD) / (denom[..., None] + 1e-16)
    return out.reshape(num, WIN, HID) + p[pre + '_bias']


def _graph_norm(x, p, pre):
    mean = x.mean(0, keepdims=True)
    out = x - p[pre + '_ms'] * mean
    var = (out * out).mean(0, keepdims=True)
    return p[pre + '_w'] * out / jnp.sqrt(var + 1e-5) + p[pre + '_b']


def _gru(x, p):
    for l in range(2):
        wih = p['gru%d_wih' % l]; whh = p['gru%d_whh' % l]
        bih = p['gru%d_bih' % l]; bhh = p['gru%d_bhh' % l]
        def step(hprev, xt):
            gi = xt @ wih.T + bih
            gh = hprev @ whh.T + bhh
            ir, iz, inn = jnp.split(gi, 3, axis=-1)
            hr, hz, hn = jnp.split(gh, 3, axis=-1)
            r = jax.nn.sigmoid(ir + hr)
            zg = jax.nn.sigmoid(iz + hz)
            n = jnp.tanh(inn + r * hn)
            hnew = (1 - zg) * n + zg * hprev
            return hnew, hnew
        h0 = jnp.zeros((x.shape[0], HID), x.dtype)
        _, ys = jax.lax.scan(step, h0, jnp.swapaxes(x, 0, 1))
        x = jnp.swapaxes(ys, 0, 1)
    return x


def _mha(x, p):
    B, T, D = x.shape
    qkv = x @ p['mha_inw'].T + p['mha_inb']
    q, k, v = jnp.split(qkv, 3, axis=-1)
    def sp(t):
        return t.reshape(B, T, H, HD).transpose(0, 2, 1, 3)
    q = sp(q); k = sp(k); v = sp(v)
    attn = jax.nn.softmax(q @ jnp.swapaxes(k, -1, -2) / jnp.sqrt(float(HD)), axis=-1)
    o = (attn @ v).transpose(0, 2, 1, 3).reshape(B, T, D)
    o = o @ p['mha_ow'].T + p['mha_ob']
    return o, attn.mean(axis=1)


def _heads_body(hf_ref, ow1, ob1, ow2, ob2, dw1, db1, dw2, db2,
                ew1, eb1, ew2, eb2, pw1, pb1, pw2, pb2, cw1, cb1, cw2, cb2,
                order_ref, demand_ref, exc_ref, prop_ref, conf_ref):
    hf = hf_ref[...]

    def mlp2(x, w1, b1, w2, b2):
        a = jnp.maximum(x @ w1[...].T + b1[...], 0.0)
        return a @ w2[...].T + b2[...]

    order_ref[...] = jnp.maximum(mlp2(hf, ow1, ob1, ow2, ob2), 0.0)
    demand_ref[...] = mlp2(hf, dw1, db1, dw2, db2)
    ex = mlp2(hf, ew1, eb1, ew2, eb2)
    ex = ex - jnp.max(ex, axis=-1, keepdims=True)
    ex = jnp.exp(ex)
    exc_ref[...] = ex / jnp.sum(ex, axis=-1, keepdims=True)
    prop_ref[...] = jax.nn.sigmoid(mlp2(hf, pw1, pb1, pw2, pb2))
    conf_ref[...] = jax.nn.sigmoid(mlp2(hf, cw1, cb1, cw2, cb2))


def _pad_head(w2, b2, fill=0.0):
    k = w2.shape[0]
    w2p = jnp.zeros((HID, w2.shape[1]), w2.dtype).at[:k].set(w2)
    b2p = jnp.full((HID,), fill, b2.dtype).at[:k].set(b2)
    return w2p, b2p


def _heads(hf, p):
    n = hf.shape[0]
    blk = 1000
    grid = n // blk
    ow2, ob2 = _pad_head(p['ord_w2'], p['ord_b2'])
    dw2, db2 = _pad_head(p['dem_w2'], p['dem_b2'])
    ew2, eb2 = _pad_head(p['exc_w2'], p['exc_b2'], fill=-1e30)
    pw2, pb2 = _pad_head(p['prp_w2'], p['prp_b2'])
    cw2, cb2 = _pad_head(p['cnf_w2'], p['cnf_b2'])
    widx = [p['ord_w1'], p['ord_b1'], ow2, ob2,
            p['dem_w1'], p['dem_b1'], dw2, db2,
            p['exc_w1'], p['exc_b1'], ew2, eb2,
            p['prp_w1'], p['prp_b1'], pw2, pb2,
            p['cnf_w1'], p['cnf_b1'], cw2, cb2]
    wspecs = [pl.BlockSpec(w.shape, functools.partial(lambda nd, i: (0,) * nd, w.ndim))
              for w in widx]
    out_shapes = [jax.ShapeDtypeStruct((n, HID), jnp.float32) for _ in range(5)]
    out_specs = [pl.BlockSpec((blk, HID), lambda i: (i, 0)) for _ in range(5)]
    order, demand, exc, prop, conf = pl.pallas_call(
        _heads_body,
        grid=(grid,),
        in_specs=[pl.BlockSpec((blk, HID), lambda i: (i, 0))] + wspecs,
        out_specs=out_specs,
        out_shape=out_shapes,
    )(hf, *widx)
    return order[:, :1], demand[:, :FH], exc[:, :3], prop[:, :FH], conf[:, :1]


def kernel(x_temporal, structural_embeddings, edge_index, edge_attr, params):
    B, W, N, _ = x_temporal.shape
    sd = structural_embeddings.shape[-1]
    struct = jnp.broadcast_to(structural_embeddings[None, None], (B, W, N, sd))
    xc = jnp.concatenate([x_temporal, struct], axis=-1)
    edge_emb = jax.nn.relu(edge_attr @ params['ee_w'].T + params['ee_b'])
    src = edge_index[0]; dst = edge_index[1]
    num = B * N
    h = jnp.einsum('tnf,kf->ntk', xc[0], params['fe_w']) + params['fe_b']
    h = _ln(h, params['fe_ln_g'], params['fe_ln_b'])
    h = jax.nn.relu(h)
    for i in range(2):
        hn = _gatv2_all(h, src, dst, edge_emb, params, 'c%d' % i, num)
        hn = _graph_norm(hn, params, 'n%d' % i)
        hn = jax.nn.relu(hn)
        h = h + hn
    ht = h.reshape(B * N, W, HID)
    hg = _gru(ht, params)
    ha, aw = _mha(hg, params)
    ht2 = _ln(hg + ha, params['tn_g'], params['tn_b'])
    hf = ht2[:, -1]
    order, demand, exc, prop, conf = _heads(hf, params)
    return (order.reshape(B, N, 1), demand.reshape(B, N, FH),
            exc.reshape(B, N, 3), prop.reshape(B, N, FH),
            conf.reshape(B, N, 1), aw.reshape(B, N, W, W))


# speedup vs baseline: 1.1876x; 1.1876x over previous
import functools

import jax
import jax.numpy as jnp
from jax import lax
from jax.experimental import pallas as pl
from jax.experimental.pallas import tpu as pltpu
from jax.experimental.pallas import tpu_sc as plsc

H = 4
HD = 32
HID = 128
FH = 4
WIN = 10

_NC = 2
_NS = 16
_NW = _NC * _NS

def _mesh():
    return plsc.VectorSubcoreMesh(
        core_axis_name="c", subcore_axis_name="s",
        num_cores=_NC, num_subcores=_NS)


def _sc_gather(table, idx, chunk):
    M = idx.shape[0]
    R, D = table.shape
    dt = table.dtype
    per_w = M // _NW
    n_pairs = per_w // (2 * chunk)

    @functools.partial(
        pl.kernel,
        mesh=_mesh(),
        out_type=jax.ShapeDtypeStruct((M, D), dt),
        scratch_types=[
            pltpu.VMEM((chunk,), jnp.int32),
            pltpu.VMEM((chunk,), jnp.int32),
            pltpu.VMEM((chunk, D), dt),
            pltpu.VMEM((chunk, D), dt),
            pltpu.SemaphoreType.DMA,
            pltpu.SemaphoreType.DMA,
        ],
    )
    def k(table_hbm, idx_hbm, out_hbm, idx_v0, idx_v1, rows_v0, rows_v1,
          sem0, sem1):
        wid = lax.axis_index("s") * _NC + lax.axis_index("c")
        base = wid * per_w

        def body(j, carry):
            off0 = base + (2 * j) * chunk
            off1 = off0 + chunk
            pltpu.sync_copy(idx_hbm.at[pl.ds(off0, chunk)], idx_v0)
            cp0 = pltpu.async_copy(table_hbm.at[idx_v0], rows_v0, sem0)
            pltpu.sync_copy(idx_hbm.at[pl.ds(off1, chunk)], idx_v1)
            cp1 = pltpu.async_copy(table_hbm.at[idx_v1], rows_v1, sem1)
            cp0.wait()
            pltpu.sync_copy(rows_v0, out_hbm.at[pl.ds(off0, chunk)])
            cp1.wait()
            pltpu.sync_copy(rows_v1, out_hbm.at[pl.ds(off1, chunk)])
            return carry

        lax.fori_loop(0, n_pairs, body, 0)

    return k(table, idx)


def _sc_segsum(upd, idx, nrows, chunk):
    M, D = upd.shape
    half = M // _NC
    per_w = half // _NS
    n_chunks = per_w // chunk
    zeros = jnp.zeros((nrows, D), jnp.float32)

    @functools.partial(
        pl.kernel,
        mesh=_mesh(),
        out_type=jax.ShapeDtypeStruct((_NC, nrows, D), jnp.float32),
        scratch_types=[
            pltpu.VMEM((chunk,), jnp.int32),
            pltpu.VMEM((chunk, D), jnp.float32),
            pltpu.VMEM_SHARED((nrows, D), jnp.float32),
        ],
    )
    def k(upd_hbm, idx_hbm, zeros_hbm, out_hbm, idx_v, upd_v, acc):
        c = lax.axis_index("c")
        s = lax.axis_index("s")
        base = c * half + s * per_w

        @pl.when(s == 0)
        def _():
            pltpu.sync_copy(zeros_hbm, acc)

        plsc.subcore_barrier()

        def body(j, carry):
            off = base + j * chunk
            pltpu.sync_copy(idx_hbm.at[pl.ds(off, chunk)], idx_v)
            pltpu.sync_copy(upd_hbm.at[pl.ds(off, chunk)], upd_v)
            pltpu.sync_copy(upd_v, acc.at[idx_v], add=True)
            return carry

        lax.fori_loop(0, n_chunks, body, 0)
        plsc.subcore_barrier()

        @pl.when(s == 0)
        def _():
            pltpu.sync_copy(acc, out_hbm.at[c])

    return k(upd, idx, zeros)


def _sc_scatter_msgs(msgs, idx, nrows, chunk):
    M = msgs.shape[0]
    half = M // _NC
    per_w = half // _NS
    n_chunks = per_w // chunk
    zeros = jnp.zeros((nrows, HID), jnp.float32)

    @functools.partial(
        pl.kernel,
        mesh=_mesh(),
        out_type=jax.ShapeDtypeStruct((_NC, nrows, WIN, HID), jnp.float32),
        scratch_types=[
            pltpu.VMEM((chunk,), jnp.int32),
            pltpu.VMEM((chunk, HID), jnp.float32),
            pltpu.VMEM_SHARED((nrows, HID), jnp.float32),
        ],
    )
    def k(msgs_hbm, idx_hbm, zeros_hbm, out_hbm, idx_v, upd_v, acc):
        c = lax.axis_index("c")
        s = lax.axis_index("s")
        base = c * half + s * per_w

        for t in range(WIN):
            @pl.when(s == 0)
            def _():
                pltpu.sync_copy(zeros_hbm, acc)

            plsc.subcore_barrier()

            def body(j, carry):
                off = base + j * chunk
                pltpu.sync_copy(idx_hbm.at[pl.ds(off, chunk)], idx_v)
                pltpu.sync_copy(
                    msgs_hbm.at[pl.ds(off, chunk), pl.ds(t * HID, HID)], upd_v)
                pltpu.sync_copy(upd_v, acc.at[idx_v], add=True)
                return carry

            lax.fori_loop(0, n_chunks, body, 0)
            plsc.subcore_barrier()

            @pl.when(s == 0)
            def _():
                pltpu.sync_copy(acc, out_hbm.at[c, :, t])

    return k(msgs, idx, zeros)


def _ln(x, g, b):
    m = x.mean(-1, keepdims=True)
    v = ((x - m) ** 2).mean(-1, keepdims=True)
    return g * (x - m) / jnp.sqrt(v + 1e-5) + b


def _gatv2_all(hs, src, dst, edge_emb, p, pre, num):
    TH = WIN * H
    xl = jnp.einsum('nwd,kd->nwk', hs, p[pre + '_ll_w']) + p[pre + '_ll_b']
    xr = jnp.einsum('nwd,kd->nwk', hs, p[pre + '_lr_w']) + p[pre + '_lr_b']
    XL = xl.reshape(num, WIN * HID)
    XR = xr.reshape(num, WIN * HID)
    XJ = _sc_gather(XL, src, 40)
    XI = _sc_gather(XR, dst, 40)
    e = edge_emb @ p[pre + '_le_w'].T
    G = jnp.repeat(jnp.eye(TH, dtype=jnp.float32), HD, axis=0)
    attv = jnp.tile(p[pre + '_att'].reshape(1, HID), (1, WIN))
    m = jax.nn.leaky_relu(XJ + XI + jnp.tile(e, (1, WIN)), 0.2)
    L = (m * attv) @ G
    ne = L.shape[0]
    P = jnp.exp(L - jnp.max(L))
    Pp = jnp.zeros((ne, 128), jnp.float32).at[:, :TH].set(P)
    S = _sc_segsum(Pp, dst, num, 80).sum(0)
    msgs = XJ * (P @ G.T)
    out = _sc_scatter_msgs(msgs, dst, num, 80).sum(0)
    denom = S[:, :TH].reshape(num, WIN, H)
    out = out.reshape(num, WIN, H, HD) / (denom[..., None] + 1e-16)
    return out.reshape(num, WIN, HID) + p[pre + '_bias']


def _graph_norm(x, p, pre):
    mean = x.mean(0, keepdims=True)
    out = x - p[pre + '_ms'] * mean
    var = (out * out).mean(0, keepdims=True)
    return p[pre + '_w'] * out / jnp.sqrt(var + 1e-5) + p[pre + '_b']


def _gru(x, p):
    for l in range(2):
        wih = p['gru%d_wih' % l]; whh = p['gru%d_whh' % l]
        bih = p['gru%d_bih' % l]; bhh = p['gru%d_bhh' % l]
        def step(hprev, xt):
            gi = xt @ wih.T + bih
            gh = hprev @ whh.T + bhh
            ir, iz, inn = jnp.split(gi, 3, axis=-1)
            hr, hz, hn = jnp.split(gh, 3, axis=-1)
            r = jax.nn.sigmoid(ir + hr)
            zg = jax.nn.sigmoid(iz + hz)
            n = jnp.tanh(inn + r * hn)
            hnew = (1 - zg) * n + zg * hprev
            return hnew, hnew
        h0 = jnp.zeros((x.shape[0], HID), x.dtype)
        _, ys = jax.lax.scan(step, h0, jnp.swapaxes(x, 0, 1))
        x = jnp.swapaxes(ys, 0, 1)
    return x


def _mha(x, p):
    B, T, D = x.shape
    qkv = x @ p['mha_inw'].T + p['mha_inb']
    q, k, v = jnp.split(qkv, 3, axis=-1)
    def sp(t):
        return t.reshape(B, T, H, HD).transpose(0, 2, 1, 3)
    q = sp(q); k = sp(k); v = sp(v)
    attn = jax.nn.softmax(q @ jnp.swapaxes(k, -1, -2) / jnp.sqrt(float(HD)), axis=-1)
    o = (attn @ v).transpose(0, 2, 1, 3).reshape(B, T, D)
    o = o @ p['mha_ow'].T + p['mha_ob']
    return o, attn.mean(axis=1)


def _heads_body(hf_ref, ow1, ob1, ow2, ob2, dw1, db1, dw2, db2,
                ew1, eb1, ew2, eb2, pw1, pb1, pw2, pb2, cw1, cb1, cw2, cb2,
                order_ref, demand_ref, exc_ref, prop_ref, conf_ref):
    hf = hf_ref[...]

    def mlp2(x, w1, b1, w2, b2):
        a = jnp.maximum(x @ w1[...].T + b1[...], 0.0)
        return a @ w2[...].T + b2[...]

    order_ref[...] = jnp.maximum(mlp2(hf, ow1, ob1, ow2, ob2), 0.0)
    demand_ref[...] = mlp2(hf, dw1, db1, dw2, db2)
    ex = mlp2(hf, ew1, eb1, ew2, eb2)
    ex = ex - jnp.max(ex, axis=-1, keepdims=True)
    ex = jnp.exp(ex)
    exc_ref[...] = ex / jnp.sum(ex, axis=-1, keepdims=True)
    prop_ref[...] = jax.nn.sigmoid(mlp2(hf, pw1, pb1, pw2, pb2))
    conf_ref[...] = jax.nn.sigmoid(mlp2(hf, cw1, cb1, cw2, cb2))


def _pad_head(w2, b2, fill=0.0):
    k = w2.shape[0]
    w2p = jnp.zeros((HID, w2.shape[1]), w2.dtype).at[:k].set(w2)
    b2p = jnp.full((HID,), fill, b2.dtype).at[:k].set(b2)
    return w2p, b2p


def _heads(hf, p):
    n = hf.shape[0]
    blk = 1000
    grid = n // blk
    ow2, ob2 = _pad_head(p['ord_w2'], p['ord_b2'])
    dw2, db2 = _pad_head(p['dem_w2'], p['dem_b2'])
    ew2, eb2 = _pad_head(p['exc_w2'], p['exc_b2'], fill=-1e30)
    pw2, pb2 = _pad_head(p['prp_w2'], p['prp_b2'])
    cw2, cb2 = _pad_head(p['cnf_w2'], p['cnf_b2'])
    widx = [p['ord_w1'], p['ord_b1'], ow2, ob2,
            p['dem_w1'], p['dem_b1'], dw2, db2,
            p['exc_w1'], p['exc_b1'], ew2, eb2,
            p['prp_w1'], p['prp_b1'], pw2, pb2,
            p['cnf_w1'], p['cnf_b1'], cw2, cb2]
    wspecs = [pl.BlockSpec(w.shape, functools.partial(lambda nd, i: (0,) * nd, w.ndim))
              for w in widx]
    out_shapes = [jax.ShapeDtypeStruct((n, HID), jnp.float32) for _ in range(5)]
    out_specs = [pl.BlockSpec((blk, HID), lambda i: (i, 0)) for _ in range(5)]
    order, demand, exc, prop, conf = pl.pallas_call(
        _heads_body,
        grid=(grid,),
        in_specs=[pl.BlockSpec((blk, HID), lambda i: (i, 0))] + wspecs,
        out_specs=out_specs,
        out_shape=out_shapes,
    )(hf, *widx)
    return order[:, :1], demand[:, :FH], exc[:, :3], prop[:, :FH], conf[:, :1]


def kernel(x_temporal, structural_embeddings, edge_index, edge_attr, params):
    B, W, N, _ = x_temporal.shape
    sd = structural_embeddings.shape[-1]
    struct = jnp.broadcast_to(structural_embeddings[None, None], (B, W, N, sd))
    xc = jnp.concatenate([x_temporal, struct], axis=-1)
    edge_emb = jax.nn.relu(edge_attr @ params['ee_w'].T + params['ee_b'])
    src = edge_index[0]; dst = edge_index[1]
    num = B * N
    h = jnp.einsum('tnf,kf->ntk', xc[0], params['fe_w']) + params['fe_b']
    h = _ln(h, params['fe_ln_g'], params['fe_ln_b'])
    h = jax.nn.relu(h)
    for i in range(2):
        hn = _gatv2_all(h, src, dst, edge_emb, params, 'c%d' % i, num)
        hn = _graph_norm(hn, params, 'n%d' % i)
        hn = jax.nn.relu(hn)
        h = h + hn
    ht = h.reshape(B * N, W, HID)
    hg = _gru(ht, params)
    ha, aw = _mha(hg, params)
    ht2 = _ln(hg + ha, params['tn_g'], params['tn_b'])
    hf = ht2[:, -1]
    order, demand, exc, prop, conf = _heads(hf, params)
    return (order.reshape(B, N, 1), demand.reshape(B, N, FH),
            exc.reshape(B, N, 3), prop.reshape(B, N, FH),
            conf.reshape(B, N, 1), aw.reshape(B, N, W, W))
